# baseline (speedup 1.0000x reference)
import functools

import jax
import jax.numpy as jnp
from jax.experimental import pallas as pl
from jax.experimental.pallas import tpu as pltpu

TILE_N = 2048


def _embed_kernel(idx_ref, day_tab_ref, week_tab_ref, out_ref, *, time_steps):
    bb = out_ref.shape[0]
    tile_n = out_ref.shape[-1]

    table = jnp.concatenate(
        [day_tab_ref[...], week_tab_ref[...]], axis=0).astype(jnp.bfloat16)

    iota_d = jax.lax.broadcasted_iota(jnp.int32, (time_steps, tile_n), 0)
    iota_w = jax.lax.broadcasted_iota(jnp.int32, (8, tile_n), 0)

    for i in range(bb):
        cmb = idx_ref[i]
        day_idx = jax.lax.shift_right_logical(cmb, 3)
        week_idx = jnp.bitwise_and(cmb, 7)

        onehot = jnp.concatenate(
            [(iota_d == day_idx).astype(jnp.bfloat16),
             (iota_w == week_idx).astype(jnp.bfloat16)], axis=0)

        res = jax.lax.dot_general(
            table, onehot, (((0,), (0,)), ((), ())),
            preferred_element_type=jnp.float32)
        out_ref[i] = res[:, None, :]


def kernel(x, time_day, time_week):
    B, T, N, C = x.shape
    time_steps, F = time_day.shape
    n_weeks = time_week.shape[0]

    week_tab = jnp.zeros((8, F), time_week.dtype).at[:n_weeks].set(time_week)

    day_idx = jnp.clip((x[:, -1:, :, 1] * float(time_steps)).astype(jnp.int32),
                       0, time_steps - 1)
    week_idx = jnp.clip(x[:, -1:, :, 2].astype(jnp.int32), 0, n_weeks - 1)
    cmb = day_idx * 8 + week_idx

    n_pad = ((N + TILE_N - 1) // TILE_N) * TILE_N
    if n_pad != N:
        cmb = jnp.pad(cmb, ((0, 0), (0, 0), (0, n_pad - N)))

    body = functools.partial(_embed_kernel, time_steps=time_steps)

    n_tiles = n_pad // TILE_N
    bb = 8 if (B % 8 == 0 and n_tiles == 1) else 1
    out = pl.pallas_call(
        body,
        out_shape=jax.ShapeDtypeStruct((B, F, 1, n_pad), jnp.float32),
        grid=(B // bb, n_tiles),
        in_specs=[
            pl.BlockSpec((bb, 1, TILE_N), lambda b, n: (b, 0, n)),
            pl.BlockSpec((time_steps, F), lambda b, n: (0, 0)),
            pl.BlockSpec((8, F), lambda b, n: (0, 0)),
        ],
        out_specs=pl.BlockSpec((bb, F, 1, TILE_N), lambda b, n: (b, 0, 0, n)),
        compiler_params=pltpu.CompilerParams(
            dimension_semantics=("parallel", "parallel")),
    )(cmb, time_day, week_tab)

    return jnp.transpose(out[:, :, :, :N], (0, 1, 3, 2))


# speedup vs baseline: 1.0337x; 1.0337x over previous
import functools

import jax
import jax.numpy as jnp
from jax.experimental import pallas as pl
from jax.experimental.pallas import tpu as pltpu

TILE_N = 2048


def _embed_kernel(day_ref, week_ref, day_tab_ref, week_tab_ref, out_ref, *,
                  time_steps, n_weeks):
    bb = out_ref.shape[0]
    tile_n = out_ref.shape[-1]

    table = jnp.concatenate(
        [day_tab_ref[...], week_tab_ref[...]], axis=0).astype(jnp.bfloat16)

    iota_d = jax.lax.broadcasted_iota(jnp.int32, (time_steps, tile_n), 0)
    iota_w = jax.lax.broadcasted_iota(jnp.int32, (8, tile_n), 0)

    for i in range(bb):
        day_idx = jnp.clip((day_ref[i] * float(time_steps)).astype(jnp.int32),
                           0, time_steps - 1)
        week_idx = jnp.clip(week_ref[i].astype(jnp.int32), 0, n_weeks - 1)

        onehot = jnp.concatenate(
            [(iota_d == day_idx).astype(jnp.bfloat16),
             (iota_w == week_idx).astype(jnp.bfloat16)], axis=0)

        res = jax.lax.dot_general(
            table, onehot, (((0,), (0,)), ((), ())),
            preferred_element_type=jnp.float32)
        out_ref[i] = res[:, None, :]


def kernel(x, time_day, time_week):
    B, T, N, C = x.shape
    time_steps, F = time_day.shape
    n_weeks = time_week.shape[0]

    week_tab = jnp.concatenate(
        [time_week, jnp.zeros((8 - n_weeks, F), time_week.dtype)], axis=0)

    day = x[:, -1:, :, 1].astype(jnp.float32)
    week = x[:, -1:, :, 2].astype(jnp.float32)
    n_pad = ((N + TILE_N - 1) // TILE_N) * TILE_N
    if n_pad != N:
        day = jnp.pad(day, ((0, 0), (0, 0), (0, n_pad - N)))
        week = jnp.pad(week, ((0, 0), (0, 0), (0, n_pad - N)))

    body = functools.partial(_embed_kernel,
                             time_steps=time_steps, n_weeks=n_weeks)

    n_tiles = n_pad // TILE_N
    bb = 8 if (B % 8 == 0 and n_tiles == 1) else 1
    out = pl.pallas_call(
        body,
        out_shape=jax.ShapeDtypeStruct((B, F, 1, n_pad), jnp.float32),
        grid=(B // bb, n_tiles),
        in_specs=[
            pl.BlockSpec((bb, 1, TILE_N), lambda b, n: (b, 0, n)),
            pl.BlockSpec((bb, 1, TILE_N), lambda b, n: (b, 0, n)),
            pl.BlockSpec((time_steps, F), lambda b, n: (0, 0)),
            pl.BlockSpec((8, F), lambda b, n: (0, 0)),
        ],
        out_specs=pl.BlockSpec((bb, F, 1, TILE_N), lambda b, n: (b, 0, 0, n)),
        compiler_params=pltpu.CompilerParams(
            dimension_semantics=("parallel", "parallel")),
    )(day, week, time_day, week_tab)

    return jnp.transpose(out[:, :, :, :N], (0, 1, 3, 2))


# speedup vs baseline: 1.1967x; 1.1577x over previous
import functools

import jax
import jax.numpy as jnp
from jax.experimental import pallas as pl
from jax.experimental.pallas import tpu as pltpu

TILE_N = 2048


def _embed_kernel(day_ref, week_ref, day_tab_ref, week_tab_ref, out_ref, *,
                  time_steps, n_weeks, t_row):
    bb = out_ref.shape[0]
    tile_n = out_ref.shape[-1]

    table = jnp.concatenate(
        [day_tab_ref[...], week_tab_ref[...]], axis=0).astype(jnp.bfloat16)

    iota_d = jax.lax.broadcasted_iota(jnp.int32, (time_steps, tile_n), 0)
    iota_w = jax.lax.broadcasted_iota(jnp.int32, (8, tile_n), 0)

    for i in range(bb):
        day = day_ref[i, t_row:t_row + 1, :]
        week = week_ref[i, t_row:t_row + 1, :]

        day_idx = jnp.clip((day * float(time_steps)).astype(jnp.int32),
                           0, time_steps - 1)
        week_idx = jnp.clip(week.astype(jnp.int32), 0, n_weeks - 1)

        onehot = jnp.concatenate(
            [(iota_d == day_idx).astype(jnp.bfloat16),
             (iota_w == week_idx).astype(jnp.bfloat16)], axis=0)

        res = jax.lax.dot_general(
            table, onehot, (((0,), (0,)), ((), ())),
            preferred_element_type=jnp.float32)
        out_ref[i] = res[:, None, :]


def kernel(x, time_day, time_week):
    B, T, N, C = x.shape
    time_steps, F = time_day.shape
    n_weeks = time_week.shape[0]

    week_tab = jnp.concatenate(
        [time_week, jnp.zeros((8 - n_weeks, F), time_week.dtype)], axis=0)

    n_pad = ((N + TILE_N - 1) // TILE_N) * TILE_N
    n_tiles = n_pad // TILE_N

    if n_pad == N and B % 8 == 0 and T % 8 == 0:
        bb = 8
        t_blk = (T - 1) // 8
        t_row = (T - 1) % 8
        xt = jnp.transpose(x.astype(jnp.float32), (0, 3, 1, 2))
        day_in = xt
        week_in = xt
        day_spec = pl.BlockSpec((bb, None, 8, TILE_N),
                                lambda b, n: (b, 1, t_blk, n))
        week_spec = pl.BlockSpec((bb, None, 8, TILE_N),
                                 lambda b, n: (b, 2, t_blk, n))
    else:
        bb = 1
        t_row = 0
        day_in = x[:, -1:, :, 1].astype(jnp.float32)
        week_in = x[:, -1:, :, 2].astype(jnp.float32)
        if n_pad != N:
            day_in = jnp.pad(day_in, ((0, 0), (0, 0), (0, n_pad - N)))
            week_in = jnp.pad(week_in, ((0, 0), (0, 0), (0, n_pad - N)))
        day_spec = pl.BlockSpec((bb, 1, TILE_N), lambda b, n: (b, 0, n))
        week_spec = pl.BlockSpec((bb, 1, TILE_N), lambda b, n: (b, 0, n))

    body = functools.partial(_embed_kernel, time_steps=time_steps,
                             n_weeks=n_weeks, t_row=t_row)

    out = pl.pallas_call(
        body,
        out_shape=jax.ShapeDtypeStruct((B, F, 1, n_pad), jnp.float32),
        grid=(B // bb, n_tiles),
        in_specs=[
            day_spec,
            week_spec,
            pl.BlockSpec((time_steps, F), lambda b, n: (0, 0)),
            pl.BlockSpec((8, F), lambda b, n: (0, 0)),
        ],
        out_specs=pl.BlockSpec((bb, F, 1, TILE_N), lambda b, n: (b, 0, 0, n)),
        compiler_params=pltpu.CompilerParams(
            dimension_semantics=("parallel", "parallel")),
    )(day_in, week_in, time_day, week_tab)

    return jnp.transpose(out[:, :, :, :N], (0, 1, 3, 2))
